# baseline (speedup 1.0000x reference)
import functools

import jax
import jax.numpy as jnp
from jax import lax
from jax.experimental import pallas as pl
from jax.experimental.pallas import tpu as pltpu
from jax.experimental.pallas import tpu_sc as plsc

_N = 10000
_C = 128
_NC = 2
_NS = 16
_NW = _NC * _NS
_RN = 640
_NP = _RN * _NS
_K = 128
_EW = 10240
_EP = _EW * _NW
_STEPS = _EW // _K
_BSTEP = 16
_NBLK = _STEPS // _BSTEP

_mesh = plsc.VectorSubcoreMesh(core_axis_name="c", subcore_axis_name="s")


def _sc_histogram(dst3, ones_rows, zeros_rows):
    @functools.partial(
        pl.kernel,
        out_type=jax.ShapeDtypeStruct((_NC, _NP, _C), jnp.float32),
        mesh=_mesh,
        scratch_types=[
            pltpu.VMEM((_BSTEP, _K), jnp.int32),
            pltpu.VMEM((_K, _C), jnp.float32),
            pltpu.VMEM_SHARED((_NP, _C), jnp.float32),
        ],
    )
    def hist(dst_hbm, ones_hbm, zeros_hbm, out_hbm, didx, ones_v, acc):
        c = lax.axis_index("c")
        s = lax.axis_index("s")
        wid = s * _NC + c
        pltpu.sync_copy(ones_hbm, ones_v)
        pltpu.sync_copy(zeros_hbm, acc.at[pl.ds(s * _RN, _RN)])
        plsc.subcore_barrier()

        @pl.loop(0, _NBLK)
        def _(blk):
            pltpu.sync_copy(
                dst_hbm.at[wid].at[pl.ds(blk * _BSTEP, _BSTEP)], didx)

            @pl.loop(0, _BSTEP)
            def _(k):
                pltpu.sync_copy(ones_v, acc.at[didx.at[k]], add=True)

        plsc.subcore_barrier()
        pltpu.sync_copy(acc.at[pl.ds(s * _RN, _RN)],
                        out_hbm.at[c].at[pl.ds(s * _RN, _RN)])

    return hist(dst3, ones_rows, zeros_rows)


def _sc_aggregate(xs, src3, dst3, zeros_rows):
    @functools.partial(
        pl.kernel,
        out_type=jax.ShapeDtypeStruct((_NC, _NP, _C), jnp.float32),
        mesh=_mesh,
        scratch_types=[
            pltpu.VMEM((_BSTEP, _K), jnp.int32),
            pltpu.VMEM((_BSTEP, _K), jnp.int32),
            pltpu.VMEM((_K, _C), jnp.float32),
            pltpu.VMEM((_K, _C), jnp.float32),
            pltpu.VMEM_SHARED((_NP, _C), jnp.float32),
            pltpu.SemaphoreType.DMA,
            pltpu.SemaphoreType.DMA,
        ],
    )
    def agg(xs_hbm, src_hbm, dst_hbm, zeros_hbm, out_hbm,
            sidx, didx, rows0, rows1, acc, sem0, sem1):
        c = lax.axis_index("c")
        s = lax.axis_index("s")
        wid = s * _NC + c
        pltpu.sync_copy(zeros_hbm, acc.at[pl.ds(s * _RN, _RN)])
        plsc.subcore_barrier()

        rows = (rows0, rows1)
        sems = (sem0, sem1)

        @pl.loop(0, _NBLK)
        def _(blk):
            pltpu.sync_copy(
                src_hbm.at[wid].at[pl.ds(blk * _BSTEP, _BSTEP)], sidx)
            pltpu.sync_copy(
                dst_hbm.at[wid].at[pl.ds(blk * _BSTEP, _BSTEP)], didx)

            pltpu.make_async_copy(xs_hbm.at[sidx.at[0]], rows0, sem0).start()
            pltpu.make_async_copy(xs_hbm.at[sidx.at[1]], rows1, sem1).start()

            @pl.loop(0, _BSTEP // 2 - 1)
            def _(j):
                k = j * 2
                for b in range(2):
                    pltpu.make_async_copy(
                        xs_hbm.at[sidx.at[k + b]], rows[b], sems[b]).wait()
                    pltpu.sync_copy(rows[b], acc.at[didx.at[k + b]], add=True)
                    pltpu.make_async_copy(
                        xs_hbm.at[sidx.at[k + 2 + b]], rows[b], sems[b]).start()

            kt = _BSTEP - 2
            for b in range(2):
                pltpu.make_async_copy(
                    xs_hbm.at[sidx.at[kt + b]], rows[b], sems[b]).wait()
                pltpu.sync_copy(rows[b], acc.at[didx.at[kt + b]], add=True)

        plsc.subcore_barrier()
        pltpu.sync_copy(acc.at[pl.ds(s * _RN, _RN)],
                        out_hbm.at[c].at[pl.ds(s * _RN, _RN)])

    return agg(xs, src3, dst3, zeros_rows)


_BR = 1280


def _tc_prescale_body(d0_ref, d1_ref, x_ref, dinv_ref, xs_ref):
    deg = d0_ref[:, 0:1] + d1_ref[:, 0:1] + 1.0
    dinv = lax.rsqrt(deg)
    dinv_b = jnp.broadcast_to(dinv, (d0_ref.shape[0], _C))
    dinv_ref[...] = dinv_b
    xs_ref[...] = dinv_b * x_ref[...]


def _tc_prescale(d0, d1, x):
    return pl.pallas_call(
        _tc_prescale_body,
        grid=(_NP // _BR,),
        in_specs=[
            pl.BlockSpec((_BR, _C), lambda i: (i, 0)),
            pl.BlockSpec((_BR, _C), lambda i: (i, 0)),
            pl.BlockSpec((_BR, _C), lambda i: (i, 0)),
        ],
        out_specs=[
            pl.BlockSpec((_BR, _C), lambda i: (i, 0)),
            pl.BlockSpec((_BR, _C), lambda i: (i, 0)),
        ],
        out_shape=[
            jax.ShapeDtypeStruct((_NP, _C), jnp.float32),
            jax.ShapeDtypeStruct((_NP, _C), jnp.float32),
        ],
    )(d0, d1, x)


def _tc_layer1_body(p0_ref, p1_ref, x_ref, dinv_ref, w_ref, b_ref,
                    h_ref, hs_ref):
    dinv = dinv_ref[...]
    g = dinv * (p0_ref[...] + p1_ref[...]) + dinv * dinv * x_ref[...]
    h = jnp.dot(g, w_ref[...], preferred_element_type=jnp.float32)
    h = jnp.maximum(h + b_ref[...], 0.0)
    h_ref[...] = h
    hs_ref[...] = dinv * h


def _tc_layer1(p0, p1, x, dinv, W1, b1):
    return pl.pallas_call(
        _tc_layer1_body,
        grid=(_NP // _BR,),
        in_specs=[
            pl.BlockSpec((_BR, _C), lambda i: (i, 0)),
            pl.BlockSpec((_BR, _C), lambda i: (i, 0)),
            pl.BlockSpec((_BR, _C), lambda i: (i, 0)),
            pl.BlockSpec((_BR, _C), lambda i: (i, 0)),
            pl.BlockSpec((_C, _C), lambda i: (0, 0)),
            pl.BlockSpec((1, _C), lambda i: (0, 0)),
        ],
        out_specs=[
            pl.BlockSpec((_BR, _C), lambda i: (i, 0)),
            pl.BlockSpec((_BR, _C), lambda i: (i, 0)),
        ],
        out_shape=[
            jax.ShapeDtypeStruct((_NP, _C), jnp.float32),
            jax.ShapeDtypeStruct((_NP, _C), jnp.float32),
        ],
    )(p0, p1, x, dinv, W1, b1)


def _tc_layer2_body(q0_ref, q1_ref, h_ref, dinv_ref, wm_ref, bm_ref,
                    wl_ref, bl_ref, mu_ref, ls_ref):
    dinv = dinv_ref[...]
    g = dinv * (q0_ref[...] + q1_ref[...]) + dinv * dinv * h_ref[...]
    mu_ref[...] = jnp.dot(g, wm_ref[...],
                          preferred_element_type=jnp.float32) + bm_ref[...]
    ls_ref[...] = jnp.dot(g, wl_ref[...],
                          preferred_element_type=jnp.float32) + bl_ref[...]


def _tc_layer2(q0, q1, h, dinv, Wmu, bmu, Wls, bls):
    return pl.pallas_call(
        _tc_layer2_body,
        grid=(_NP // _BR,),
        in_specs=[
            pl.BlockSpec((_BR, _C), lambda i: (i, 0)),
            pl.BlockSpec((_BR, _C), lambda i: (i, 0)),
            pl.BlockSpec((_BR, _C), lambda i: (i, 0)),
            pl.BlockSpec((_BR, _C), lambda i: (i, 0)),
            pl.BlockSpec((_C, _C), lambda i: (0, 0)),
            pl.BlockSpec((1, _C), lambda i: (0, 0)),
            pl.BlockSpec((_C, _C), lambda i: (0, 0)),
            pl.BlockSpec((1, _C), lambda i: (0, 0)),
        ],
        out_specs=[
            pl.BlockSpec((_BR, _C), lambda i: (i, 0)),
            pl.BlockSpec((_BR, _C), lambda i: (i, 0)),
        ],
        out_shape=[
            jax.ShapeDtypeStruct((_NP, _C), jnp.float32),
            jax.ShapeDtypeStruct((_NP, _C), jnp.float32),
        ],
    )(q0, q1, h, dinv, Wmu, bmu, Wls, bls)


def kernel(x, edge_index, W1, b1, Wmu, bmu, Wls, bls):
    E = edge_index.shape[1]
    pad = _EP - E
    fill = jnp.full((pad,), _N, jnp.int32)
    src3 = jnp.concatenate([edge_index[0].astype(jnp.int32), fill])
    dst3 = jnp.concatenate([edge_index[1].astype(jnp.int32), fill])
    src3 = src3.reshape(_NW, _STEPS, _K)
    dst3 = dst3.reshape(_NW, _STEPS, _K)
    xp = jnp.pad(x, ((0, _NP - _N), (0, 0)))

    ones_rows = jnp.ones((_K, _C), jnp.float32)
    zeros_rows = jnp.zeros((_RN, _C), jnp.float32)
    b1r = b1.reshape(1, _C)
    bmur = bmu.reshape(1, _C)
    blsr = bls.reshape(1, _C)

    degp = _sc_histogram(dst3, ones_rows, zeros_rows)
    dinv, xs = _tc_prescale(degp[0], degp[1], xp)
    p = _sc_aggregate(xs, src3, dst3, zeros_rows)
    h, hs = _tc_layer1(p[0], p[1], xp, dinv, W1, b1r)
    q = _sc_aggregate(hs, src3, dst3, zeros_rows)
    mu, ls = _tc_layer2(q[0], q[1], h, dinv, Wmu, bmur, Wls, blsr)
    return (mu[:_N], ls[:_N])


# speedup vs baseline: 2.7000x; 2.7000x over previous
import functools

import jax
import jax.numpy as jnp
from jax import lax
from jax.experimental import pallas as pl
from jax.experimental.pallas import tpu as pltpu
from jax.experimental.pallas import tpu_sc as plsc

_N = 10000
_C = 128
_NC = 2
_NS = 16
_NW = _NC * _NS
_RN = 640
_NP = _RN * _NS
_K = 128
_EW = 10240
_EP = _EW * _NW
_STEPS = _EW // _K
_BSTEP = 16
_NBLK = _STEPS // _BSTEP

_mesh = plsc.VectorSubcoreMesh(core_axis_name="c", subcore_axis_name="s")


def _sc_histogram(dst3, ones_rows, zeros_rows):
    @functools.partial(
        pl.kernel,
        out_type=jax.ShapeDtypeStruct((_NC, _NP, _C), jnp.float32),
        mesh=_mesh,
        scratch_types=[
            pltpu.VMEM((_BSTEP, _K), jnp.int32),
            pltpu.VMEM((_K, _C), jnp.float32),
            pltpu.VMEM_SHARED((_NP, _C), jnp.float32),
        ],
    )
    def hist(dst_hbm, ones_hbm, zeros_hbm, out_hbm, didx, ones_v, acc):
        c = lax.axis_index("c")
        s = lax.axis_index("s")
        wid = s * _NC + c
        pltpu.sync_copy(ones_hbm, ones_v)
        pltpu.sync_copy(zeros_hbm, acc.at[pl.ds(s * _RN, _RN)])
        plsc.subcore_barrier()

        @pl.loop(0, _NBLK)
        def _(blk):
            pltpu.sync_copy(
                dst_hbm.at[wid].at[pl.ds(blk * _BSTEP, _BSTEP)], didx)

            @pl.loop(0, _BSTEP)
            def _(k):
                pltpu.sync_copy(ones_v, acc.at[didx.at[k]], add=True)

        plsc.subcore_barrier()
        pltpu.sync_copy(acc.at[pl.ds(s * _RN, _RN)],
                        out_hbm.at[c].at[pl.ds(s * _RN, _RN)])

    return hist(dst3, ones_rows, zeros_rows)


def _sc_aggregate(xs, src3, dst3, zeros_rows):
    @functools.partial(
        pl.kernel,
        out_type=jax.ShapeDtypeStruct((_NC, _NP, _C), jnp.float32),
        mesh=_mesh,
        scratch_types=[
            pltpu.VMEM((_BSTEP, _K), jnp.int32),
            pltpu.VMEM((_BSTEP, _K), jnp.int32),
            pltpu.VMEM((_K, _C), jnp.float32),
            pltpu.VMEM((_K, _C), jnp.float32),
            pltpu.VMEM_SHARED((_NP, _C), jnp.float32),
            pltpu.SemaphoreType.DMA,
            pltpu.SemaphoreType.DMA,
        ],
    )
    def agg(xs_hbm, src_hbm, dst_hbm, zeros_hbm, out_hbm,
            sidx, didx, rows0, rows1, acc, sem0, sem1):
        c = lax.axis_index("c")
        s = lax.axis_index("s")
        wid = s * _NC + c
        pltpu.sync_copy(zeros_hbm, acc.at[pl.ds(s * _RN, _RN)])
        plsc.subcore_barrier()

        rows = (rows0, rows1)
        sems = (sem0, sem1)

        @pl.loop(0, _NBLK)
        def _(blk):
            pltpu.sync_copy(
                src_hbm.at[wid].at[pl.ds(blk * _BSTEP, _BSTEP)], sidx)
            pltpu.sync_copy(
                dst_hbm.at[wid].at[pl.ds(blk * _BSTEP, _BSTEP)], didx)

            pltpu.make_async_copy(xs_hbm.at[sidx.at[0]], rows0, sem0).start()
            pltpu.make_async_copy(xs_hbm.at[sidx.at[1]], rows1, sem1).start()

            @pl.loop(0, _BSTEP // 2 - 1)
            def _(j):
                k = j * 2
                for b in range(2):
                    pltpu.make_async_copy(
                        xs_hbm.at[sidx.at[k + b]], rows[b], sems[b]).wait()
                    pltpu.sync_copy(rows[b], acc.at[didx.at[k + b]], add=True)
                    pltpu.make_async_copy(
                        xs_hbm.at[sidx.at[k + 2 + b]], rows[b], sems[b]).start()

            kt = _BSTEP - 2
            for b in range(2):
                pltpu.make_async_copy(
                    xs_hbm.at[sidx.at[kt + b]], rows[b], sems[b]).wait()
                pltpu.sync_copy(rows[b], acc.at[didx.at[kt + b]], add=True)

        plsc.subcore_barrier()
        pltpu.sync_copy(acc.at[pl.ds(s * _RN, _RN)],
                        out_hbm.at[c].at[pl.ds(s * _RN, _RN)])

    return agg(xs, src3, dst3, zeros_rows)


_BR = 1280


def _tc_prescale_body(d0_ref, d1_ref, x_ref, dinv_ref, xs_ref):
    deg = d0_ref[:, 0:1] + d1_ref[:, 0:1] + 1.0
    dinv = lax.rsqrt(deg)
    dinv_b = jnp.broadcast_to(dinv, (d0_ref.shape[0], _C))
    dinv_ref[...] = dinv_b
    xs_ref[...] = dinv_b * x_ref[...]


def _tc_prescale(d0, d1, x):
    return pl.pallas_call(
        _tc_prescale_body,
        grid=(_NP // _BR,),
        in_specs=[
            pl.BlockSpec((_BR, _C), lambda i: (i, 0)),
            pl.BlockSpec((_BR, _C), lambda i: (i, 0)),
            pl.BlockSpec((_BR, _C), lambda i: (i, 0)),
        ],
        out_specs=[
            pl.BlockSpec((_BR, _C), lambda i: (i, 0)),
            pl.BlockSpec((_BR, _C), lambda i: (i, 0)),
        ],
        out_shape=[
            jax.ShapeDtypeStruct((_NP, _C), jnp.float32),
            jax.ShapeDtypeStruct((_NP, _C), jnp.float32),
        ],
    )(d0, d1, x)


def _tc_layer1_body(p0_ref, p1_ref, x_ref, dinv_ref, w_ref, b_ref,
                    h_ref, hs_ref):
    dinv = dinv_ref[...]
    g = dinv * (p0_ref[...] + p1_ref[...]) + dinv * dinv * x_ref[...]
    h = jnp.dot(g, w_ref[...], preferred_element_type=jnp.float32)
    h = jnp.maximum(h + b_ref[...], 0.0)
    h_ref[...] = h
    hs_ref[...] = dinv * h


def _tc_layer1(p0, p1, x, dinv, W1, b1):
    return pl.pallas_call(
        _tc_layer1_body,
        grid=(_NP // _BR,),
        in_specs=[
            pl.BlockSpec((_BR, _C), lambda i: (i, 0)),
            pl.BlockSpec((_BR, _C), lambda i: (i, 0)),
            pl.BlockSpec((_BR, _C), lambda i: (i, 0)),
            pl.BlockSpec((_BR, _C), lambda i: (i, 0)),
            pl.BlockSpec((_C, _C), lambda i: (0, 0)),
            pl.BlockSpec((1, _C), lambda i: (0, 0)),
        ],
        out_specs=[
            pl.BlockSpec((_BR, _C), lambda i: (i, 0)),
            pl.BlockSpec((_BR, _C), lambda i: (i, 0)),
        ],
        out_shape=[
            jax.ShapeDtypeStruct((_NP, _C), jnp.float32),
            jax.ShapeDtypeStruct((_NP, _C), jnp.float32),
        ],
    )(p0, p1, x, dinv, W1, b1)


def _tc_layer2_body(q0_ref, q1_ref, h_ref, dinv_ref, wm_ref, bm_ref,
                    wl_ref, bl_ref, mu_ref, ls_ref):
    dinv = dinv_ref[...]
    g = dinv * (q0_ref[...] + q1_ref[...]) + dinv * dinv * h_ref[...]
    mu_ref[...] = jnp.dot(g, wm_ref[...],
                          preferred_element_type=jnp.float32) + bm_ref[...]
    ls_ref[...] = jnp.dot(g, wl_ref[...],
                          preferred_element_type=jnp.float32) + bl_ref[...]


def _tc_layer2(q0, q1, h, dinv, Wmu, bmu, Wls, bls):
    return pl.pallas_call(
        _tc_layer2_body,
        grid=(_NP // _BR,),
        in_specs=[
            pl.BlockSpec((_BR, _C), lambda i: (i, 0)),
            pl.BlockSpec((_BR, _C), lambda i: (i, 0)),
            pl.BlockSpec((_BR, _C), lambda i: (i, 0)),
            pl.BlockSpec((_BR, _C), lambda i: (i, 0)),
            pl.BlockSpec((_C, _C), lambda i: (0, 0)),
            pl.BlockSpec((1, _C), lambda i: (0, 0)),
            pl.BlockSpec((_C, _C), lambda i: (0, 0)),
            pl.BlockSpec((1, _C), lambda i: (0, 0)),
        ],
        out_specs=[
            pl.BlockSpec((_BR, _C), lambda i: (i, 0)),
            pl.BlockSpec((_BR, _C), lambda i: (i, 0)),
        ],
        out_shape=[
            jax.ShapeDtypeStruct((_NP, _C), jnp.float32),
            jax.ShapeDtypeStruct((_NP, _C), jnp.float32),
        ],
    )(q0, q1, h, dinv, Wmu, bmu, Wls, bls)


def kernel(x, edge_index, W1, b1, Wmu, bmu, Wls, bls):
    E = edge_index.shape[1]
    pad = _EP - E
    fill = _N + (jnp.arange(pad, dtype=jnp.int32) % (_NP - _N))
    src3 = jnp.concatenate([edge_index[0].astype(jnp.int32), fill])
    dst3 = jnp.concatenate([edge_index[1].astype(jnp.int32), fill])
    src3 = src3.reshape(_NW, _STEPS, _K)
    dst3 = dst3.reshape(_NW, _STEPS, _K)
    xp = jnp.pad(x, ((0, _NP - _N), (0, 0)))

    ones_rows = jnp.ones((_K, _C), jnp.float32)
    zeros_rows = jnp.zeros((_RN, _C), jnp.float32)
    b1r = b1.reshape(1, _C)
    bmur = bmu.reshape(1, _C)
    blsr = bls.reshape(1, _C)

    degp = _sc_histogram(dst3, ones_rows, zeros_rows)
    dinv, xs = _tc_prescale(degp[0], degp[1], xp)
    p = _sc_aggregate(xs, src3, dst3, zeros_rows)
    h, hs = _tc_layer1(p[0], p[1], xp, dinv, W1, b1r)
    q = _sc_aggregate(hs, src3, dst3, zeros_rows)
    mu, ls = _tc_layer2(q[0], q[1], h, dinv, Wmu, bmur, Wls, blsr)
    return (mu[:_N], ls[:_N])


# speedup vs baseline: 2.7675x; 1.0250x over previous
import functools

import jax
import jax.numpy as jnp
from jax import lax
from jax.experimental import pallas as pl
from jax.experimental.pallas import tpu as pltpu
from jax.experimental.pallas import tpu_sc as plsc

_N = 10000
_C = 128
_NC = 2
_NS = 16
_NW = _NC * _NS
_RN = 640
_NP = _RN * _NS
_K = 128
_EW = 10240
_EP = _EW * _NW
_STEPS = _EW // _K
_BSTEP = 16
_NBLK = _STEPS // _BSTEP

_mesh = plsc.VectorSubcoreMesh(core_axis_name="c", subcore_axis_name="s")


def _sc_histogram(dst3, ones_rows, zeros_rows):
    @functools.partial(
        pl.kernel,
        out_type=jax.ShapeDtypeStruct((_NC, _NP, _C), jnp.float32),
        mesh=_mesh,
        scratch_types=[
            pltpu.VMEM((_BSTEP, _K), jnp.int32),
            pltpu.VMEM((_K, _C), jnp.float32),
            pltpu.VMEM_SHARED((_NP, _C), jnp.float32),
        ],
    )
    def hist(dst_hbm, ones_hbm, zeros_hbm, out_hbm, didx, ones_v, acc):
        c = lax.axis_index("c")
        s = lax.axis_index("s")
        wid = s * _NC + c
        pltpu.sync_copy(ones_hbm, ones_v)
        pltpu.sync_copy(zeros_hbm, acc.at[pl.ds(s * _RN, _RN)])
        plsc.subcore_barrier()

        @pl.loop(0, _NBLK)
        def _(blk):
            pltpu.sync_copy(
                dst_hbm.at[wid].at[pl.ds(blk * _BSTEP, _BSTEP)], didx)

            @pl.loop(0, _BSTEP)
            def _(k):
                pltpu.sync_copy(ones_v, acc.at[didx.at[k]], add=True)

        plsc.subcore_barrier()
        pltpu.sync_copy(acc.at[pl.ds(s * _RN, _RN)],
                        out_hbm.at[c].at[pl.ds(s * _RN, _RN)])

    return hist(dst3, ones_rows, zeros_rows)


def _sc_aggregate(xs, src3, dst3, zeros_rows):
    @functools.partial(
        pl.kernel,
        out_type=jax.ShapeDtypeStruct((_NC, _NP, _C), jnp.float32),
        mesh=_mesh,
        scratch_types=[
            pltpu.VMEM((_BSTEP, _K), jnp.int32),
            pltpu.VMEM((_BSTEP, _K), jnp.int32),
            pltpu.VMEM((_K, _C), jnp.float32),
            pltpu.VMEM((_K, _C), jnp.float32),
            pltpu.VMEM_SHARED((_NP, _C), jnp.float32),
            pltpu.SemaphoreType.DMA,
            pltpu.SemaphoreType.DMA,
        ],
    )
    def agg(xs_hbm, src_hbm, dst_hbm, zeros_hbm, out_hbm,
            sidx, didx, rows0, rows1, acc, sem0, sem1):
        c = lax.axis_index("c")
        s = lax.axis_index("s")
        wid = s * _NC + c
        pltpu.sync_copy(zeros_hbm, acc.at[pl.ds(s * _RN, _RN)])
        plsc.subcore_barrier()

        rows = (rows0, rows1)
        sems = (sem0, sem1)

        @pl.loop(0, _NBLK)
        def _(blk):
            pltpu.sync_copy(
                src_hbm.at[wid].at[pl.ds(blk * _BSTEP, _BSTEP)], sidx)
            pltpu.sync_copy(
                dst_hbm.at[wid].at[pl.ds(blk * _BSTEP, _BSTEP)], didx)

            pltpu.make_async_copy(xs_hbm.at[sidx.at[0]], rows0, sem0).start()
            pltpu.make_async_copy(xs_hbm.at[sidx.at[1]], rows1, sem1).start()

            @pl.loop(0, _BSTEP // 2 - 1)
            def _(j):
                k = j * 2
                for b in range(2):
                    pltpu.make_async_copy(
                        xs_hbm.at[sidx.at[k + b]], rows[b], sems[b]).wait()
                    pltpu.sync_copy(rows[b], acc.at[didx.at[k + b]], add=True)
                    pltpu.make_async_copy(
                        xs_hbm.at[sidx.at[k + 2 + b]], rows[b], sems[b]).start()

            kt = _BSTEP - 2
            for b in range(2):
                pltpu.make_async_copy(
                    xs_hbm.at[sidx.at[kt + b]], rows[b], sems[b]).wait()
                pltpu.sync_copy(rows[b], acc.at[didx.at[kt + b]], add=True)

        plsc.subcore_barrier()
        pltpu.sync_copy(acc.at[pl.ds(s * _RN, _RN)],
                        out_hbm.at[c].at[pl.ds(s * _RN, _RN)])

    return agg(xs, src3, dst3, zeros_rows)


_BR = 1280


def _tc_prescale_body(d0_ref, d1_ref, x_ref, dinv_ref, xs_ref):
    deg = d0_ref[:, 0:1] + d1_ref[:, 0:1] + 1.0
    dinv = lax.rsqrt(deg)
    dinv_b = jnp.broadcast_to(dinv, (d0_ref.shape[0], _C))
    dinv_ref[...] = dinv_b
    xs_ref[...] = dinv_b * x_ref[...]


def _tc_prescale(d0, d1, x):
    return pl.pallas_call(
        _tc_prescale_body,
        grid=(_NP // _BR,),
        in_specs=[
            pl.BlockSpec((_BR, _C), lambda i: (i, 0)),
            pl.BlockSpec((_BR, _C), lambda i: (i, 0)),
            pl.BlockSpec((_BR, _C), lambda i: (i, 0)),
        ],
        out_specs=[
            pl.BlockSpec((_BR, _C), lambda i: (i, 0)),
            pl.BlockSpec((_BR, _C), lambda i: (i, 0)),
        ],
        out_shape=[
            jax.ShapeDtypeStruct((_NP, _C), jnp.float32),
            jax.ShapeDtypeStruct((_NP, _C), jnp.float32),
        ],
    )(d0, d1, x)


def _tc_layer1_body(p0_ref, p1_ref, x_ref, dinv_ref, w_ref, b_ref,
                    h_ref, hs_ref):
    dinv = dinv_ref[...]
    g = dinv * (p0_ref[...] + p1_ref[...]) + dinv * dinv * x_ref[...]
    h = jnp.dot(g, w_ref[...], preferred_element_type=jnp.float32)
    h = jnp.maximum(h + b_ref[...], 0.0)
    h_ref[...] = h
    hs_ref[...] = dinv * h


def _tc_layer1(p0, p1, x, dinv, W1, b1):
    return pl.pallas_call(
        _tc_layer1_body,
        grid=(_NP // _BR,),
        in_specs=[
            pl.BlockSpec((_BR, _C), lambda i: (i, 0)),
            pl.BlockSpec((_BR, _C), lambda i: (i, 0)),
            pl.BlockSpec((_BR, _C), lambda i: (i, 0)),
            pl.BlockSpec((_BR, _C), lambda i: (i, 0)),
            pl.BlockSpec((_C, _C), lambda i: (0, 0)),
            pl.BlockSpec((1, _C), lambda i: (0, 0)),
        ],
        out_specs=[
            pl.BlockSpec((_BR, _C), lambda i: (i, 0)),
            pl.BlockSpec((_BR, _C), lambda i: (i, 0)),
        ],
        out_shape=[
            jax.ShapeDtypeStruct((_NP, _C), jnp.float32),
            jax.ShapeDtypeStruct((_NP, _C), jnp.float32),
        ],
    )(p0, p1, x, dinv, W1, b1)


def _tc_layer2_body(q0_ref, q1_ref, h_ref, dinv_ref, wm_ref, bm_ref,
                    wl_ref, bl_ref, mu_ref, ls_ref):
    dinv = dinv_ref[...]
    g = dinv * (q0_ref[...] + q1_ref[...]) + dinv * dinv * h_ref[...]
    mu_ref[...] = jnp.dot(g, wm_ref[...],
                          preferred_element_type=jnp.float32) + bm_ref[...]
    ls_ref[...] = jnp.dot(g, wl_ref[...],
                          preferred_element_type=jnp.float32) + bl_ref[...]


_BR2 = 2000


def _tc_layer2(q0, q1, h, dinv, Wmu, bmu, Wls, bls):
    return pl.pallas_call(
        _tc_layer2_body,
        grid=(_N // _BR2,),
        in_specs=[
            pl.BlockSpec((_BR2, _C), lambda i: (i, 0)),
            pl.BlockSpec((_BR2, _C), lambda i: (i, 0)),
            pl.BlockSpec((_BR2, _C), lambda i: (i, 0)),
            pl.BlockSpec((_BR2, _C), lambda i: (i, 0)),
            pl.BlockSpec((_C, _C), lambda i: (0, 0)),
            pl.BlockSpec((1, _C), lambda i: (0, 0)),
            pl.BlockSpec((_C, _C), lambda i: (0, 0)),
            pl.BlockSpec((1, _C), lambda i: (0, 0)),
        ],
        out_specs=[
            pl.BlockSpec((_BR2, _C), lambda i: (i, 0)),
            pl.BlockSpec((_BR2, _C), lambda i: (i, 0)),
        ],
        out_shape=[
            jax.ShapeDtypeStruct((_N, _C), jnp.float32),
            jax.ShapeDtypeStruct((_N, _C), jnp.float32),
        ],
    )(q0, q1, h, dinv, Wmu, bmu, Wls, bls)


def kernel(x, edge_index, W1, b1, Wmu, bmu, Wls, bls):
    E = edge_index.shape[1]
    pad = _EP - E
    fill = _N + (jnp.arange(pad, dtype=jnp.int32) % (_NP - _N))
    src3 = jnp.concatenate([edge_index[0].astype(jnp.int32), fill])
    dst3 = jnp.concatenate([edge_index[1].astype(jnp.int32), fill])
    src3 = src3.reshape(_NW, _STEPS, _K)
    dst3 = dst3.reshape(_NW, _STEPS, _K)
    xp = jnp.pad(x, ((0, _NP - _N), (0, 0)))

    ones_rows = jnp.ones((_K, _C), jnp.float32)
    zeros_rows = jnp.zeros((_RN, _C), jnp.float32)
    b1r = b1.reshape(1, _C)
    bmur = bmu.reshape(1, _C)
    blsr = bls.reshape(1, _C)

    degp = _sc_histogram(dst3, ones_rows, zeros_rows)
    dinv, xs = _tc_prescale(degp[0], degp[1], xp)
    p = _sc_aggregate(xs, src3, dst3, zeros_rows)
    h, hs = _tc_layer1(p[0], p[1], xp, dinv, W1, b1r)
    q = _sc_aggregate(hs, src3, dst3, zeros_rows)
    mu, ls = _tc_layer2(q[0], q[1], h, dinv, Wmu, bmur, Wls, blsr)
    return (mu, ls)


# speedup vs baseline: 2.8934x; 1.0455x over previous
import functools

import jax
import jax.numpy as jnp
from jax import lax
from jax.experimental import pallas as pl
from jax.experimental.pallas import tpu as pltpu
from jax.experimental.pallas import tpu_sc as plsc

_N = 10000
_C = 128
_NC = 2
_NS = 16
_NW = _NC * _NS
_RN = 640
_NP = _RN * _NS
_K = 128
_EW = 10240
_EP = _EW * _NW
_STEPS = _EW // _K
_BSTEP = 40
_NBLK = _STEPS // _BSTEP

_mesh = plsc.VectorSubcoreMesh(core_axis_name="c", subcore_axis_name="s")


def _sc_histogram(dst3, ones_rows, zeros_rows):
    @functools.partial(
        pl.kernel,
        out_type=jax.ShapeDtypeStruct((_NC, _NP, _C), jnp.float32),
        mesh=_mesh,
        scratch_types=[
            pltpu.VMEM((_BSTEP, _K), jnp.int32),
            pltpu.VMEM((_K, _C), jnp.float32),
            pltpu.VMEM_SHARED((_NP, _C), jnp.float32),
        ],
    )
    def hist(dst_hbm, ones_hbm, zeros_hbm, out_hbm, didx, ones_v, acc):
        c = lax.axis_index("c")
        s = lax.axis_index("s")
        wid = s * _NC + c
        pltpu.sync_copy(ones_hbm, ones_v)
        pltpu.sync_copy(zeros_hbm, acc.at[pl.ds(s * _RN, _RN)])
        plsc.subcore_barrier()

        @pl.loop(0, _NBLK)
        def _(blk):
            pltpu.sync_copy(
                dst_hbm.at[wid].at[pl.ds(blk * _BSTEP, _BSTEP)], didx)

            @pl.loop(0, _BSTEP)
            def _(k):
                pltpu.sync_copy(ones_v, acc.at[didx.at[k]], add=True)

        plsc.subcore_barrier()
        pltpu.sync_copy(acc.at[pl.ds(s * _RN, _RN)],
                        out_hbm.at[c].at[pl.ds(s * _RN, _RN)])

    return hist(dst3, ones_rows, zeros_rows)


def _sc_aggregate(xs, src3, dst3, zeros_rows):
    @functools.partial(
        pl.kernel,
        out_type=jax.ShapeDtypeStruct((_NC, _NP, _C), jnp.float32),
        mesh=_mesh,
        scratch_types=[
            pltpu.VMEM((_BSTEP, _K), jnp.int32),
            pltpu.VMEM((_BSTEP, _K), jnp.int32),
            pltpu.VMEM((_K, _C), jnp.float32),
            pltpu.VMEM((_K, _C), jnp.float32),
            pltpu.VMEM_SHARED((_NP, _C), jnp.float32),
            pltpu.SemaphoreType.DMA,
            pltpu.SemaphoreType.DMA,
        ],
    )
    def agg(xs_hbm, src_hbm, dst_hbm, zeros_hbm, out_hbm,
            sidx, didx, rows0, rows1, acc, sem0, sem1):
        c = lax.axis_index("c")
        s = lax.axis_index("s")
        wid = s * _NC + c
        pltpu.sync_copy(zeros_hbm, acc.at[pl.ds(s * _RN, _RN)])
        plsc.subcore_barrier()

        rows = (rows0, rows1)
        sems = (sem0, sem1)

        @pl.loop(0, _NBLK)
        def _(blk):
            pltpu.sync_copy(
                src_hbm.at[wid].at[pl.ds(blk * _BSTEP, _BSTEP)], sidx)
            pltpu.sync_copy(
                dst_hbm.at[wid].at[pl.ds(blk * _BSTEP, _BSTEP)], didx)

            pltpu.make_async_copy(xs_hbm.at[sidx.at[0]], rows0, sem0).start()
            pltpu.make_async_copy(xs_hbm.at[sidx.at[1]], rows1, sem1).start()

            @pl.loop(0, _BSTEP // 2 - 1)
            def _(j):
                k = j * 2
                for b in range(2):
                    pltpu.make_async_copy(
                        xs_hbm.at[sidx.at[k + b]], rows[b], sems[b]).wait()
                    pltpu.sync_copy(rows[b], acc.at[didx.at[k + b]], add=True)
                    pltpu.make_async_copy(
                        xs_hbm.at[sidx.at[k + 2 + b]], rows[b], sems[b]).start()

            kt = _BSTEP - 2
            for b in range(2):
                pltpu.make_async_copy(
                    xs_hbm.at[sidx.at[kt + b]], rows[b], sems[b]).wait()
                pltpu.sync_copy(rows[b], acc.at[didx.at[kt + b]], add=True)

        plsc.subcore_barrier()
        pltpu.sync_copy(acc.at[pl.ds(s * _RN, _RN)],
                        out_hbm.at[c].at[pl.ds(s * _RN, _RN)])

    return agg(xs, src3, dst3, zeros_rows)


_BR = 1280


def _tc_prescale_body(d0_ref, d1_ref, x_ref, dinv_ref, xs_ref):
    deg = d0_ref[:, 0:1] + d1_ref[:, 0:1] + 1.0
    dinv = lax.rsqrt(deg)
    dinv_b = jnp.broadcast_to(dinv, (d0_ref.shape[0], _C))
    dinv_ref[...] = dinv_b
    xs_ref[...] = dinv_b * x_ref[...]


def _tc_prescale(d0, d1, x):
    return pl.pallas_call(
        _tc_prescale_body,
        grid=(_NP // _BR,),
        in_specs=[
            pl.BlockSpec((_BR, _C), lambda i: (i, 0)),
            pl.BlockSpec((_BR, _C), lambda i: (i, 0)),
            pl.BlockSpec((_BR, _C), lambda i: (i, 0)),
        ],
        out_specs=[
            pl.BlockSpec((_BR, _C), lambda i: (i, 0)),
            pl.BlockSpec((_BR, _C), lambda i: (i, 0)),
        ],
        out_shape=[
            jax.ShapeDtypeStruct((_NP, _C), jnp.float32),
            jax.ShapeDtypeStruct((_NP, _C), jnp.float32),
        ],
    )(d0, d1, x)


def _tc_layer1_body(p0_ref, p1_ref, x_ref, dinv_ref, w_ref, b_ref,
                    h_ref, hs_ref):
    dinv = dinv_ref[...]
    g = dinv * (p0_ref[...] + p1_ref[...]) + dinv * dinv * x_ref[...]
    h = jnp.dot(g, w_ref[...], preferred_element_type=jnp.float32)
    h = jnp.maximum(h + b_ref[...], 0.0)
    h_ref[...] = h
    hs_ref[...] = dinv * h


def _tc_layer1(p0, p1, x, dinv, W1, b1):
    return pl.pallas_call(
        _tc_layer1_body,
        grid=(_NP // _BR,),
        in_specs=[
            pl.BlockSpec((_BR, _C), lambda i: (i, 0)),
            pl.BlockSpec((_BR, _C), lambda i: (i, 0)),
            pl.BlockSpec((_BR, _C), lambda i: (i, 0)),
            pl.BlockSpec((_BR, _C), lambda i: (i, 0)),
            pl.BlockSpec((_C, _C), lambda i: (0, 0)),
            pl.BlockSpec((1, _C), lambda i: (0, 0)),
        ],
        out_specs=[
            pl.BlockSpec((_BR, _C), lambda i: (i, 0)),
            pl.BlockSpec((_BR, _C), lambda i: (i, 0)),
        ],
        out_shape=[
            jax.ShapeDtypeStruct((_NP, _C), jnp.float32),
            jax.ShapeDtypeStruct((_NP, _C), jnp.float32),
        ],
    )(p0, p1, x, dinv, W1, b1)


def _tc_layer2_body(q0_ref, q1_ref, h_ref, dinv_ref, wm_ref, bm_ref,
                    wl_ref, bl_ref, mu_ref, ls_ref):
    dinv = dinv_ref[...]
    g = dinv * (q0_ref[...] + q1_ref[...]) + dinv * dinv * h_ref[...]
    mu_ref[...] = jnp.dot(g, wm_ref[...],
                          preferred_element_type=jnp.float32) + bm_ref[...]
    ls_ref[...] = jnp.dot(g, wl_ref[...],
                          preferred_element_type=jnp.float32) + bl_ref[...]


_BR2 = 2000


def _tc_layer2(q0, q1, h, dinv, Wmu, bmu, Wls, bls):
    return pl.pallas_call(
        _tc_layer2_body,
        grid=(_N // _BR2,),
        in_specs=[
            pl.BlockSpec((_BR2, _C), lambda i: (i, 0)),
            pl.BlockSpec((_BR2, _C), lambda i: (i, 0)),
            pl.BlockSpec((_BR2, _C), lambda i: (i, 0)),
            pl.BlockSpec((_BR2, _C), lambda i: (i, 0)),
            pl.BlockSpec((_C, _C), lambda i: (0, 0)),
            pl.BlockSpec((1, _C), lambda i: (0, 0)),
            pl.BlockSpec((_C, _C), lambda i: (0, 0)),
            pl.BlockSpec((1, _C), lambda i: (0, 0)),
        ],
        out_specs=[
            pl.BlockSpec((_BR2, _C), lambda i: (i, 0)),
            pl.BlockSpec((_BR2, _C), lambda i: (i, 0)),
        ],
        out_shape=[
            jax.ShapeDtypeStruct((_N, _C), jnp.float32),
            jax.ShapeDtypeStruct((_N, _C), jnp.float32),
        ],
    )(q0, q1, h, dinv, Wmu, bmu, Wls, bls)


def kernel(x, edge_index, W1, b1, Wmu, bmu, Wls, bls):
    E = edge_index.shape[1]
    pad = _EP - E
    fill = _N + (jnp.arange(pad, dtype=jnp.int32) % (_NP - _N))
    src3 = jnp.concatenate([edge_index[0].astype(jnp.int32), fill])
    dst3 = jnp.concatenate([edge_index[1].astype(jnp.int32), fill])
    src3 = src3.reshape(_NW, _STEPS, _K)
    dst3 = dst3.reshape(_NW, _STEPS, _K)
    xp = jnp.pad(x, ((0, _NP - _N), (0, 0)))

    ones_rows = jnp.ones((_K, _C), jnp.float32)
    zeros_rows = jnp.zeros((_RN, _C), jnp.float32)
    b1r = b1.reshape(1, _C)
    bmur = bmu.reshape(1, _C)
    blsr = bls.reshape(1, _C)

    degp = _sc_histogram(dst3, ones_rows, zeros_rows)
    dinv, xs = _tc_prescale(degp[0], degp[1], xp)
    p = _sc_aggregate(xs, src3, dst3, zeros_rows)
    h, hs = _tc_layer1(p[0], p[1], xp, dinv, W1, b1r)
    q = _sc_aggregate(hs, src3, dst3, zeros_rows)
    mu, ls = _tc_layer2(q[0], q[1], h, dinv, Wmu, bmur, Wls, blsr)
    return (mu, ls)


# speedup vs baseline: 2.9562x; 1.0217x over previous
import functools

import jax
import jax.numpy as jnp
from jax import lax
from jax.experimental import pallas as pl
from jax.experimental.pallas import tpu as pltpu
from jax.experimental.pallas import tpu_sc as plsc

_N = 10000
_C = 128
_NC = 2
_NS = 16
_NW = _NC * _NS
_RN = 640
_NP = _RN * _NS
_K = 64
_EW = 10240
_EP = _EW * _NW
_STEPS = _EW // _K
_BSTEP = 40
_NBLK = _STEPS // _BSTEP
_NBUF = 4

_mesh = plsc.VectorSubcoreMesh(core_axis_name="c", subcore_axis_name="s")


def _sc_histogram(dst3, ones_rows, zeros_rows):
    @functools.partial(
        pl.kernel,
        out_type=jax.ShapeDtypeStruct((_NC, _NP, _C), jnp.float32),
        mesh=_mesh,
        scratch_types=[
            pltpu.VMEM((_BSTEP, _K), jnp.int32),
            pltpu.VMEM((_K, _C), jnp.float32),
            pltpu.VMEM_SHARED((_NP, _C), jnp.float32),
        ],
    )
    def hist(dst_hbm, ones_hbm, zeros_hbm, out_hbm, didx, ones_v, acc):
        c = lax.axis_index("c")
        s = lax.axis_index("s")
        wid = s * _NC + c
        pltpu.sync_copy(ones_hbm, ones_v)
        pltpu.sync_copy(zeros_hbm, acc.at[pl.ds(s * _RN, _RN)])
        plsc.subcore_barrier()

        @pl.loop(0, _NBLK)
        def _(blk):
            pltpu.sync_copy(
                dst_hbm.at[wid].at[pl.ds(blk * _BSTEP, _BSTEP)], didx)

            @pl.loop(0, _BSTEP)
            def _(k):
                pltpu.sync_copy(ones_v, acc.at[didx.at[k]], add=True)

        plsc.subcore_barrier()
        pltpu.sync_copy(acc.at[pl.ds(s * _RN, _RN)],
                        out_hbm.at[c].at[pl.ds(s * _RN, _RN)])

    return hist(dst3, ones_rows, zeros_rows)


def _sc_aggregate(xs, src3, dst3, zeros_rows):
    @functools.partial(
        pl.kernel,
        out_type=jax.ShapeDtypeStruct((_NC, _NP, _C), jnp.float32),
        mesh=_mesh,
        scratch_types=[
            pltpu.VMEM((_BSTEP, _K), jnp.int32),
            pltpu.VMEM((_BSTEP, _K), jnp.int32),
        ] + [pltpu.VMEM((_K, _C), jnp.float32) for _ in range(_NBUF)] + [
            pltpu.VMEM_SHARED((_NP, _C), jnp.float32),
        ] + [pltpu.SemaphoreType.DMA for _ in range(_NBUF)],
    )
    def agg(xs_hbm, src_hbm, dst_hbm, zeros_hbm, out_hbm,
            sidx, didx, *rest):
        rows = rest[:_NBUF]
        acc = rest[_NBUF]
        sems = rest[_NBUF + 1:]
        c = lax.axis_index("c")
        s = lax.axis_index("s")
        wid = s * _NC + c
        pltpu.sync_copy(zeros_hbm, acc.at[pl.ds(s * _RN, _RN)])
        plsc.subcore_barrier()

        @pl.loop(0, _NBLK)
        def _(blk):
            pltpu.sync_copy(
                src_hbm.at[wid].at[pl.ds(blk * _BSTEP, _BSTEP)], sidx)
            pltpu.sync_copy(
                dst_hbm.at[wid].at[pl.ds(blk * _BSTEP, _BSTEP)], didx)

            for b in range(_NBUF):
                pltpu.make_async_copy(
                    xs_hbm.at[sidx.at[b]], rows[b], sems[b]).start()

            @pl.loop(0, _BSTEP // _NBUF - 1)
            def _(j):
                k = j * _NBUF
                for b in range(_NBUF):
                    pltpu.make_async_copy(
                        xs_hbm.at[sidx.at[k + b]], rows[b], sems[b]).wait()
                    pltpu.sync_copy(rows[b], acc.at[didx.at[k + b]], add=True)
                    pltpu.make_async_copy(
                        xs_hbm.at[sidx.at[k + _NBUF + b]],
                        rows[b], sems[b]).start()

            kt = _BSTEP - _NBUF
            for b in range(_NBUF):
                pltpu.make_async_copy(
                    xs_hbm.at[sidx.at[kt + b]], rows[b], sems[b]).wait()
                pltpu.sync_copy(rows[b], acc.at[didx.at[kt + b]], add=True)

        plsc.subcore_barrier()
        pltpu.sync_copy(acc.at[pl.ds(s * _RN, _RN)],
                        out_hbm.at[c].at[pl.ds(s * _RN, _RN)])

    return agg(xs, src3, dst3, zeros_rows)


_BR = 1280


def _tc_prescale_body(d0_ref, d1_ref, x_ref, dinv_ref, xs_ref):
    deg = d0_ref[:, 0:1] + d1_ref[:, 0:1] + 1.0
    dinv = lax.rsqrt(deg)
    dinv_b = jnp.broadcast_to(dinv, (d0_ref.shape[0], _C))
    dinv_ref[...] = dinv_b
    xs_ref[...] = dinv_b * x_ref[...]


def _tc_prescale(d0, d1, x):
    return pl.pallas_call(
        _tc_prescale_body,
        grid=(_NP // _BR,),
        in_specs=[
            pl.BlockSpec((_BR, _C), lambda i: (i, 0)),
            pl.BlockSpec((_BR, _C), lambda i: (i, 0)),
            pl.BlockSpec((_BR, _C), lambda i: (i, 0)),
        ],
        out_specs=[
            pl.BlockSpec((_BR, _C), lambda i: (i, 0)),
            pl.BlockSpec((_BR, _C), lambda i: (i, 0)),
        ],
        out_shape=[
            jax.ShapeDtypeStruct((_NP, _C), jnp.float32),
            jax.ShapeDtypeStruct((_NP, _C), jnp.float32),
        ],
    )(d0, d1, x)


def _tc_layer1_body(p0_ref, p1_ref, x_ref, dinv_ref, w_ref, b_ref,
                    h_ref, hs_ref):
    dinv = dinv_ref[...]
    g = dinv * (p0_ref[...] + p1_ref[...]) + dinv * dinv * x_ref[...]
    h = jnp.dot(g, w_ref[...], preferred_element_type=jnp.float32)
    h = jnp.maximum(h + b_ref[...], 0.0)
    h_ref[...] = h
    hs_ref[...] = dinv * h


def _tc_layer1(p0, p1, x, dinv, W1, b1):
    return pl.pallas_call(
        _tc_layer1_body,
        grid=(_NP // _BR,),
        in_specs=[
            pl.BlockSpec((_BR, _C), lambda i: (i, 0)),
            pl.BlockSpec((_BR, _C), lambda i: (i, 0)),
            pl.BlockSpec((_BR, _C), lambda i: (i, 0)),
            pl.BlockSpec((_BR, _C), lambda i: (i, 0)),
            pl.BlockSpec((_C, _C), lambda i: (0, 0)),
            pl.BlockSpec((1, _C), lambda i: (0, 0)),
        ],
        out_specs=[
            pl.BlockSpec((_BR, _C), lambda i: (i, 0)),
            pl.BlockSpec((_BR, _C), lambda i: (i, 0)),
        ],
        out_shape=[
            jax.ShapeDtypeStruct((_NP, _C), jnp.float32),
            jax.ShapeDtypeStruct((_NP, _C), jnp.float32),
        ],
    )(p0, p1, x, dinv, W1, b1)


def _tc_layer2_body(q0_ref, q1_ref, h_ref, dinv_ref, wm_ref, bm_ref,
                    wl_ref, bl_ref, mu_ref, ls_ref):
    dinv = dinv_ref[...]
    g = dinv * (q0_ref[...] + q1_ref[...]) + dinv * dinv * h_ref[...]
    mu_ref[...] = jnp.dot(g, wm_ref[...],
                          preferred_element_type=jnp.float32) + bm_ref[...]
    ls_ref[...] = jnp.dot(g, wl_ref[...],
                          preferred_element_type=jnp.float32) + bl_ref[...]


_BR2 = 2000


def _tc_layer2(q0, q1, h, dinv, Wmu, bmu, Wls, bls):
    return pl.pallas_call(
        _tc_layer2_body,
        grid=(_N // _BR2,),
        in_specs=[
            pl.BlockSpec((_BR2, _C), lambda i: (i, 0)),
            pl.BlockSpec((_BR2, _C), lambda i: (i, 0)),
            pl.BlockSpec((_BR2, _C), lambda i: (i, 0)),
            pl.BlockSpec((_BR2, _C), lambda i: (i, 0)),
            pl.BlockSpec((_C, _C), lambda i: (0, 0)),
            pl.BlockSpec((1, _C), lambda i: (0, 0)),
            pl.BlockSpec((_C, _C), lambda i: (0, 0)),
            pl.BlockSpec((1, _C), lambda i: (0, 0)),
        ],
        out_specs=[
            pl.BlockSpec((_BR2, _C), lambda i: (i, 0)),
            pl.BlockSpec((_BR2, _C), lambda i: (i, 0)),
        ],
        out_shape=[
            jax.ShapeDtypeStruct((_N, _C), jnp.float32),
            jax.ShapeDtypeStruct((_N, _C), jnp.float32),
        ],
    )(q0, q1, h, dinv, Wmu, bmu, Wls, bls)


def kernel(x, edge_index, W1, b1, Wmu, bmu, Wls, bls):
    E = edge_index.shape[1]
    pad = _EP - E
    fill = _N + (jnp.arange(pad, dtype=jnp.int32) % (_NP - _N))
    src3 = jnp.concatenate([edge_index[0].astype(jnp.int32), fill])
    dst3 = jnp.concatenate([edge_index[1].astype(jnp.int32), fill])
    src3 = src3.reshape(_NW, _STEPS, _K)
    dst3 = dst3.reshape(_NW, _STEPS, _K)
    xp = jnp.pad(x, ((0, _NP - _N), (0, 0)))

    ones_rows = jnp.ones((_K, _C), jnp.float32)
    zeros_rows = jnp.zeros((_RN, _C), jnp.float32)
    b1r = b1.reshape(1, _C)
    bmur = bmu.reshape(1, _C)
    blsr = bls.reshape(1, _C)

    degp = _sc_histogram(dst3, ones_rows, zeros_rows)
    dinv, xs = _tc_prescale(degp[0], degp[1], xp)
    p = _sc_aggregate(xs, src3, dst3, zeros_rows)
    h, hs = _tc_layer1(p[0], p[1], xp, dinv, W1, b1r)
    q = _sc_aggregate(hs, src3, dst3, zeros_rows)
    mu, ls = _tc_layer2(q[0], q[1], h, dinv, Wmu, bmur, Wls, blsr)
    return (mu, ls)
